# baseline (speedup 1.0000x reference)
import functools

import jax
import jax.numpy as jnp
from jax import lax
from jax.experimental import pallas as pl
from jax.experimental.pallas import tpu as pltpu
from jax.experimental.pallas import tpu_sc as plsc

NUM_SEGMENTS = 64
NUM_ROWS = 10000
NUM_COLS = 128
CHUNK = 80
NUM_CHUNKS = NUM_ROWS // CHUNK
NUM_WORKERS = 32
MAX_CHUNKS_PER_WORKER = -(-NUM_CHUNKS // NUM_WORKERS)


def _sc_partials(x, batch):
    mesh = plsc.VectorSubcoreMesh(core_axis_name="c", subcore_axis_name="s")

    @functools.partial(
        pl.kernel,
        out_type=jax.ShapeDtypeStruct((2, NUM_SEGMENTS, NUM_COLS), jnp.float32),
        mesh=mesh,
        scratch_types=[
            pltpu.VMEM((MAX_CHUNKS_PER_WORKER, CHUNK), jnp.int32),
            pltpu.VMEM((MAX_CHUNKS_PER_WORKER, CHUNK, NUM_COLS), jnp.float32),
            pltpu.VMEM((NUM_SEGMENTS // 16, NUM_COLS), jnp.float32),
            pltpu.VMEM_SHARED((NUM_SEGMENTS, NUM_COLS), jnp.float32),
            [pltpu.SemaphoreType.DMA] * MAX_CHUNKS_PER_WORKER,
        ],
    )
    def k(x_hbm, batch_hbm, part_hbm, idx_v, rows_v, zero_v, acc_sh, sems):
        cid = lax.axis_index("c")
        sid = lax.axis_index("s")
        wid = sid * 2 + cid

        def start(k_, c):
            base = c * CHUNK
            pltpu.async_copy(batch_hbm.at[pl.ds(base, CHUNK)], idx_v.at[k_], sems[k_])
            pltpu.async_copy(x_hbm.at[pl.ds(base, CHUNK)], rows_v.at[k_], sems[k_])

        def wait(k_, c):
            base = c * CHUNK
            pltpu.make_async_copy(
                batch_hbm.at[pl.ds(base, CHUNK)], idx_v.at[k_], sems[k_]).wait()
            pltpu.make_async_copy(
                x_hbm.at[pl.ds(base, CHUNK)], rows_v.at[k_], sems[k_]).wait()

        for k_ in range(MAX_CHUNKS_PER_WORKER):
            c = wid + NUM_WORKERS * k_

            @pl.when(c < NUM_CHUNKS)
            def _():
                start(k_, c)

        zrows = NUM_SEGMENTS // 16
        for r in range(zrows):
            for cb in range(NUM_COLS // 16):
                zero_v[r, pl.ds(cb * 16, 16)] = jnp.zeros((16,), jnp.float32)
        pltpu.sync_copy(zero_v, acc_sh.at[pl.ds(sid * zrows, zrows)])
        plsc.subcore_barrier()

        for k_ in range(MAX_CHUNKS_PER_WORKER):
            c = wid + NUM_WORKERS * k_

            @pl.when(c < NUM_CHUNKS)
            def _():
                wait(k_, c)
                pltpu.sync_copy(rows_v.at[k_], acc_sh.at[idx_v.at[k_]], add=True)

        plsc.subcore_barrier()

        pltpu.sync_copy(acc_sh.at[pl.ds(sid * zrows, zrows)],
                        part_hbm.at[cid, pl.ds(sid * zrows, zrows)])

    return k(x, batch)


def _combine(p_ref, o1_ref, o2_ref):
    s = p_ref[0] + p_ref[1]
    o1_ref[...] = s
    o2_ref[...] = s


def kernel(x, edge_index, batch, train_mask):
    del edge_index, train_mask
    partials = _sc_partials(x, batch)
    out_sds = jax.ShapeDtypeStruct((NUM_SEGMENTS, NUM_COLS), jnp.float32)
    m1, m2 = pl.pallas_call(_combine, out_shape=(out_sds, out_sds))(partials)
    return (m1, m2)


# speedup vs baseline: 1.0057x; 1.0057x over previous
import functools

import jax
import jax.numpy as jnp
from jax import lax
from jax.experimental import pallas as pl
from jax.experimental.pallas import tpu as pltpu
from jax.experimental.pallas import tpu_sc as plsc

NUM_SEGMENTS = 64
NUM_ROWS = 10000
NUM_COLS = 128
CHUNK = 80
NUM_CHUNKS = NUM_ROWS // CHUNK
NUM_WORKERS = 32
MAX_CHUNKS_PER_WORKER = -(-NUM_CHUNKS // NUM_WORKERS)


def _sc_partials(x, batch):
    mesh = plsc.VectorSubcoreMesh(core_axis_name="c", subcore_axis_name="s")

    @functools.partial(
        pl.kernel,
        out_type=jax.ShapeDtypeStruct((2, NUM_SEGMENTS, NUM_COLS), jnp.float32),
        mesh=mesh,
        scratch_types=[
            pltpu.VMEM((MAX_CHUNKS_PER_WORKER, CHUNK), jnp.int32),
            pltpu.VMEM((MAX_CHUNKS_PER_WORKER, CHUNK, NUM_COLS), jnp.float32),
            pltpu.VMEM((NUM_SEGMENTS // 16, NUM_COLS), jnp.float32),
            pltpu.VMEM_SHARED((NUM_SEGMENTS, NUM_COLS), jnp.float32),
            [pltpu.SemaphoreType.DMA] * MAX_CHUNKS_PER_WORKER,
            pltpu.SemaphoreType.DMA,
        ],
    )
    def k(x_hbm, batch_hbm, part_hbm, idx_v, rows_v, zero_v, acc_sh, sems, sem_s):
        cid = lax.axis_index("c")
        sid = lax.axis_index("s")
        wid = sid * 2 + cid

        def start(k_, c):
            base = c * CHUNK
            pltpu.async_copy(batch_hbm.at[pl.ds(base, CHUNK)], idx_v.at[k_], sems[k_])
            pltpu.async_copy(x_hbm.at[pl.ds(base, CHUNK)], rows_v.at[k_], sems[k_])

        def wait(k_, c):
            base = c * CHUNK
            pltpu.make_async_copy(
                batch_hbm.at[pl.ds(base, CHUNK)], idx_v.at[k_], sems[k_]).wait()
            pltpu.make_async_copy(
                x_hbm.at[pl.ds(base, CHUNK)], rows_v.at[k_], sems[k_]).wait()

        for k_ in range(MAX_CHUNKS_PER_WORKER):
            c = wid + NUM_WORKERS * k_

            @pl.when(c < NUM_CHUNKS)
            def _():
                start(k_, c)

        zrows = NUM_SEGMENTS // 16
        for r in range(zrows):
            for cb in range(NUM_COLS // 16):
                zero_v[r, pl.ds(cb * 16, 16)] = jnp.zeros((16,), jnp.float32)
        pltpu.sync_copy(zero_v, acc_sh.at[pl.ds(sid * zrows, zrows)])
        plsc.subcore_barrier()

        for k_ in range(MAX_CHUNKS_PER_WORKER):
            c = wid + NUM_WORKERS * k_

            @pl.when(c < NUM_CHUNKS)
            def _():
                wait(k_, c)
                pltpu.async_copy(rows_v.at[k_], acc_sh.at[idx_v.at[k_]], sem_s,
                                 add=True)

        for k_ in range(MAX_CHUNKS_PER_WORKER):
            c = wid + NUM_WORKERS * k_

            @pl.when(c < NUM_CHUNKS)
            def _():
                pltpu.make_async_copy(
                    rows_v.at[k_], acc_sh.at[idx_v.at[k_]], sem_s).wait()

        plsc.subcore_barrier()

        pltpu.sync_copy(acc_sh.at[pl.ds(sid * zrows, zrows)],
                        part_hbm.at[cid, pl.ds(sid * zrows, zrows)])

    return k(x, batch)


def _combine(p_ref, o1_ref, o2_ref):
    s = p_ref[0] + p_ref[1]
    o1_ref[...] = s
    o2_ref[...] = s


def kernel(x, edge_index, batch, train_mask):
    del edge_index, train_mask
    partials = _sc_partials(x, batch)
    out_sds = jax.ShapeDtypeStruct((NUM_SEGMENTS, NUM_COLS), jnp.float32)
    m1, m2 = pl.pallas_call(_combine, out_shape=(out_sds, out_sds))(partials)
    return (m1, m2)
